# baseline (speedup 1.0000x reference)
import functools

import jax
import jax.numpy as jnp
from jax import lax
from jax.experimental import pallas as pl
from jax.experimental.pallas import tpu as pltpu, tpu_sc as plsc

N_NODES = 10000
N_EDGES = 320000
D_FEAT = 128

_NC = 2
_NS = 16
_NW = _NC * _NS
_LANES = 16

_CHUNK = N_EDGES // _NW
_K = 96
_WIN = 8
_NWIN = 14
_NB = _NWIN * _WIN
_CHUNK_PAD = _NB * _K
_N_PAD = 10112
_DUMP_ROW = _N_PAD - 1
_ROWS_PER_TILE = _N_PAD // _NS

_mesh = plsc.VectorSubcoreMesh(core_axis_name="c", subcore_axis_name="s")


def _fill_2d(ref, nrows, ncols, val):
    v = jnp.full((_LANES,), val, jnp.float32)

    def row(i, _):
        for j in range(ncols // _LANES):
            ref[i, pl.ds(j * _LANES, _LANES)] = v
        return 0

    lax.fori_loop(0, nrows, row, 0)


@functools.partial(
    pl.kernel,
    out_type=(
        jax.ShapeDtypeStruct((_NC, _N_PAD, D_FEAT), jnp.float32),
        jax.ShapeDtypeStruct((_NC, _N_PAD, D_FEAT), jnp.float32),
    ),
    mesh=_mesh,
    scratch_types=(
        pltpu.VMEM((2, _WIN, _K), jnp.int32),
        pltpu.VMEM((2, _WIN, _K), jnp.int32),
        pltpu.VMEM((_K, D_FEAT), jnp.float32),
        pltpu.VMEM((_K, D_FEAT), jnp.float32),
        pltpu.VMEM_SHARED((_N_PAD, D_FEAT), jnp.float32),
        pltpu.SemaphoreType.DMA,
        pltpu.SemaphoreType.DMA,
        pltpu.SemaphoreType.DMA,
        pltpu.SemaphoreType.DMA,
    ),
)
def _phase1(feat_hbm, nidx_hbm, seg_hbm, sums_out, cnts_out,
            idx_v, seg_v, rows0, rows1, acc, sem0, sem1, sem_i, sem_s):
    cid = lax.axis_index("c")
    sid = lax.axis_index("s")
    wid = cid * _NS + sid
    r0 = sid * _ROWS_PER_TILE
    nzb = _ROWS_PER_TILE // _K
    nzt = _ROWS_PER_TILE - nzb * _K
    rows = (rows0, rows1)
    gsems = (sem0, sem1)

    def zero_acc():
        for i in range(nzb):
            pltpu.sync_copy(rows0, acc.at[pl.ds(r0 + i * _K, _K)])
        pltpu.sync_copy(rows0.at[pl.ds(0, nzt)],
                        acc.at[pl.ds(r0 + nzb * _K, nzt)])

    _fill_2d(rows0, _K, D_FEAT, 0.0)
    zero_acc()
    plsc.subcore_barrier()

    pltpu.sync_copy(nidx_hbm.at[wid, 0], idx_v.at[0])
    pltpu.sync_copy(seg_hbm.at[wid, 0], seg_v.at[0])
    pltpu.async_copy(feat_hbm.at[idx_v.at[0, 0]], rows0, sem0)

    def window_a(w, _):
        slot = lax.rem(w, 2)
        nslot = lax.rem(w + 1, 2)

        @pl.when(w + 1 < _NWIN)
        def _():
            pltpu.async_copy(nidx_hbm.at[wid, w + 1], idx_v.at[nslot], sem_i)
            pltpu.async_copy(seg_hbm.at[wid, w + 1], seg_v.at[nslot], sem_s)

        for b in range(_WIN):
            cur = b % 2
            nxt = (b + 1) % 2
            if b + 1 < _WIN:
                pltpu.async_copy(feat_hbm.at[idx_v.at[slot, b + 1]],
                                 rows[nxt], gsems[nxt])
            pltpu.make_async_copy(feat_hbm.at[idx_v.at[slot, b]],
                                  rows[cur], gsems[cur]).wait()
            pltpu.sync_copy(rows[cur], acc.at[seg_v.at[slot, b]], add=True)

        @pl.when(w + 1 < _NWIN)
        def _():
            pltpu.make_async_copy(nidx_hbm.at[wid, w + 1], idx_v.at[nslot],
                                  sem_i).wait()
            pltpu.make_async_copy(seg_hbm.at[wid, w + 1], seg_v.at[nslot],
                                  sem_s).wait()
            pltpu.async_copy(feat_hbm.at[idx_v.at[nslot, 0]], rows0, sem0)

        return 0

    lax.fori_loop(0, _NWIN, window_a, 0)
    plsc.subcore_barrier()
    pltpu.sync_copy(acc.at[pl.ds(r0, _ROWS_PER_TILE)],
                    sums_out.at[cid, pl.ds(r0, _ROWS_PER_TILE)])
    plsc.subcore_barrier()

    _fill_2d(rows0, _K, D_FEAT, 0.0)
    zero_acc()
    _fill_2d(rows0, _K, D_FEAT, 1.0)
    plsc.subcore_barrier()

    pltpu.sync_copy(seg_hbm.at[wid, 0], seg_v.at[0])

    def window_b(w, _):
        slot = lax.rem(w, 2)
        nslot = lax.rem(w + 1, 2)

        @pl.when(w + 1 < _NWIN)
        def _():
            pltpu.async_copy(seg_hbm.at[wid, w + 1], seg_v.at[nslot], sem_s)

        for b in range(_WIN):
            pltpu.sync_copy(rows0, acc.at[seg_v.at[slot, b]], add=True)

        @pl.when(w + 1 < _NWIN)
        def _():
            pltpu.make_async_copy(seg_hbm.at[wid, w + 1], seg_v.at[nslot],
                                  sem_s).wait()

        return 0

    lax.fori_loop(0, _NWIN, window_b, 0)
    plsc.subcore_barrier()
    pltpu.sync_copy(acc.at[pl.ds(r0, _ROWS_PER_TILE)],
                    cnts_out.at[cid, pl.ds(r0, _ROWS_PER_TILE)])


_BLK = 2000


def _combine_body(sums_ref, cnts_ref, out_ref):
    s = sums_ref[0] + sums_ref[1]
    c = cnts_ref[0] + cnts_ref[1]
    out_ref[...] = jnp.where(c > 0.0, s / jnp.maximum(c, 1.0), 0.0)


_combine = pl.pallas_call(
    _combine_body,
    grid=(N_NODES // _BLK,),
    in_specs=[
        pl.BlockSpec((_NC, _BLK, D_FEAT), lambda i: (0, i, 0)),
        pl.BlockSpec((_NC, _BLK, 1), lambda i: (0, i, 0)),
    ],
    out_specs=pl.BlockSpec((_BLK, D_FEAT), lambda i: (i, 0)),
    out_shape=jax.ShapeDtypeStruct((N_NODES, D_FEAT), jnp.float32),
)


def kernel(features, neighbor_idx, segment_ids, num_samples):
    del num_samples
    pad = _CHUNK_PAD - _CHUNK
    nidx = jnp.pad(neighbor_idx.reshape(_NW, _CHUNK), ((0, 0), (0, pad)),
                   constant_values=0).reshape(_NW, _NWIN, _WIN, _K)
    seg = jnp.pad(segment_ids.reshape(_NW, _CHUNK), ((0, 0), (0, pad)),
                  constant_values=_DUMP_ROW).reshape(_NW, _NWIN, _WIN, _K)
    sums, cnts = _phase1(features, nidx, seg)
    cnts_col = cnts[:, :N_NODES, 0:1]
    return _combine(sums, cnts_col)


# speedup vs baseline: 3.0963x; 3.0963x over previous
import functools

import jax
import jax.numpy as jnp
from jax import lax
from jax.experimental import pallas as pl
from jax.experimental.pallas import tpu as pltpu, tpu_sc as plsc

N_NODES = 10000
N_EDGES = 320000
D_FEAT = 128

_NC = 2
_NS = 16
_NW = _NC * _NS
_LANES = 16

_CHUNK = N_EDGES // _NW
_K = 80
_NB = 126
_CHUNK_PAD = _NB * _K
_N_PAD = 10112
_DUMP_ROW = _N_PAD - 1
_ROWS_PER_TILE = _N_PAD // _NS

_mesh = plsc.VectorSubcoreMesh(core_axis_name="c", subcore_axis_name="s")


def _fill_2d(ref, nrows, ncols, val):
    v = jnp.full((_LANES,), val, jnp.float32)

    def row(i, _):
        for j in range(ncols // _LANES):
            ref[i, pl.ds(j * _LANES, _LANES)] = v
        return 0

    lax.fori_loop(0, nrows, row, 0)


@functools.partial(
    pl.kernel,
    out_type=(
        jax.ShapeDtypeStruct((_NC, _N_PAD, D_FEAT), jnp.float32),
        jax.ShapeDtypeStruct((_NC, _N_PAD, D_FEAT), jnp.float32),
    ),
    mesh=_mesh,
    scratch_types=(
        pltpu.VMEM((_K,), jnp.int32),
        pltpu.VMEM((_K,), jnp.int32),
        pltpu.VMEM((_K,), jnp.int32),
        pltpu.VMEM((_K,), jnp.int32),
        pltpu.VMEM((_K,), jnp.int32),
        pltpu.VMEM((_K,), jnp.int32),
        pltpu.VMEM((_K, D_FEAT), jnp.float32),
        pltpu.VMEM((_K, D_FEAT), jnp.float32),
        pltpu.VMEM((_K, D_FEAT), jnp.float32),
        pltpu.VMEM_SHARED((_N_PAD, D_FEAT), jnp.float32),
        pltpu.SemaphoreType.DMA,
        pltpu.SemaphoreType.DMA,
        pltpu.SemaphoreType.DMA,
        pltpu.SemaphoreType.DMA,
        pltpu.SemaphoreType.DMA,
        pltpu.SemaphoreType.DMA,
    ),
)
def _phase1(feat_hbm, nidx_hbm, seg_hbm, sums_out, cnts_out,
            idx0, idx1, idx2, seg0, seg1, seg2, rows0, rows1, rows2, acc,
            gs0, gs1, gs2, ss0, ss1, ss2):
    cid = lax.axis_index("c")
    sid = lax.axis_index("s")
    wid = cid * _NS + sid
    base = wid * _CHUNK_PAD
    r0 = sid * _ROWS_PER_TILE
    nzb = _ROWS_PER_TILE // _K
    nzt = _ROWS_PER_TILE - nzb * _K
    idxb = (idx0, idx1, idx2)
    segb = (seg0, seg1, seg2)
    rows = (rows0, rows1, rows2)
    gsem = (gs0, gs1, gs2)
    ssem = (ss0, ss1, ss2)

    def zero_acc():
        for i in range(nzb):
            pltpu.sync_copy(rows0, acc.at[pl.ds(r0 + i * _K, _K)])
        pltpu.sync_copy(rows0.at[pl.ds(0, nzt)],
                        acc.at[pl.ds(r0 + nzb * _K, nzt)])

    def wait_scatter(b):
        pltpu.make_async_copy(rows[b], acc.at[segb[b]], ssem[b]).wait()

    def wait_gather(b, g):
        pltpu.make_async_copy(feat_hbm.at[idxb[b]], rows[b], gsem[b]).wait()

    _fill_2d(rows0, _K, D_FEAT, 0.0)
    zero_acc()
    plsc.subcore_barrier()

    for b in range(2):
        pltpu.sync_copy(nidx_hbm.at[pl.ds(base + b * _K, _K)], idxb[b])
        pltpu.async_copy(feat_hbm.at[idxb[b]], rows[b], gsem[b])

    def triple_a(i, _):
        for b in range(3):
            g = 3 * i + b
            b2 = (b + 2) % 3
            pred = jnp.logical_and(g >= 1, g + 2 < _NB)

            @pl.when(pred)
            def _():
                wait_scatter(b2)

            @pl.when(g + 2 < _NB)
            def _():
                pltpu.sync_copy(nidx_hbm.at[pl.ds(base + (g + 2) * _K, _K)],
                                idxb[b2])
                pltpu.async_copy(feat_hbm.at[idxb[b2]], rows[b2], gsem[b2])

            wait_gather(b, g)
            pltpu.sync_copy(seg_hbm.at[pl.ds(base + g * _K, _K)], segb[b])
            pltpu.async_copy(rows[b], acc.at[segb[b]], ssem[b], add=True)
        return 0

    lax.fori_loop(0, _NB // 3, triple_a, 0)
    for g in range(_NB - 3, _NB):
        wait_scatter(g % 3)
    plsc.subcore_barrier()
    pltpu.sync_copy(acc.at[pl.ds(r0, _ROWS_PER_TILE)],
                    sums_out.at[cid, pl.ds(r0, _ROWS_PER_TILE)])
    plsc.subcore_barrier()

    _fill_2d(rows0, _K, D_FEAT, 0.0)
    zero_acc()
    _fill_2d(rows0, _K, D_FEAT, 1.0)
    plsc.subcore_barrier()

    def wait_scatter_b(b):
        pltpu.make_async_copy(rows0, acc.at[segb[b]], ssem[b]).wait()

    def triple_b(i, _):
        for b in range(3):
            g = 3 * i + b

            @pl.when(g >= 3)
            def _():
                wait_scatter_b(b)

            pltpu.sync_copy(seg_hbm.at[pl.ds(base + g * _K, _K)], segb[b])
            pltpu.async_copy(rows0, acc.at[segb[b]], ssem[b], add=True)
        return 0

    lax.fori_loop(0, _NB // 3, triple_b, 0)
    for b in range(3):
        wait_scatter_b(b)
    plsc.subcore_barrier()
    pltpu.sync_copy(acc.at[pl.ds(r0, _ROWS_PER_TILE)],
                    cnts_out.at[cid, pl.ds(r0, _ROWS_PER_TILE)])


_BLK = 2000


def _combine_body(sums_ref, cnts_ref, out_ref):
    s = sums_ref[0] + sums_ref[1]
    c = cnts_ref[0] + cnts_ref[1]
    out_ref[...] = jnp.where(c > 0.0, s / jnp.maximum(c, 1.0), 0.0)


_combine = pl.pallas_call(
    _combine_body,
    grid=(N_NODES // _BLK,),
    in_specs=[
        pl.BlockSpec((_NC, _BLK, D_FEAT), lambda i: (0, i, 0)),
        pl.BlockSpec((_NC, _BLK, 1), lambda i: (0, i, 0)),
    ],
    out_specs=pl.BlockSpec((_BLK, D_FEAT), lambda i: (i, 0)),
    out_shape=jax.ShapeDtypeStruct((N_NODES, D_FEAT), jnp.float32),
)


def kernel(features, neighbor_idx, segment_ids, num_samples):
    del num_samples
    pad = _CHUNK_PAD - _CHUNK
    nidx = jnp.pad(neighbor_idx.reshape(_NW, _CHUNK), ((0, 0), (0, pad)),
                   constant_values=0).reshape(_NW * _CHUNK_PAD)
    seg = jnp.pad(segment_ids.reshape(_NW, _CHUNK), ((0, 0), (0, pad)),
                  constant_values=_DUMP_ROW).reshape(_NW * _CHUNK_PAD)
    sums, cnts = _phase1(features, nidx, seg)
    cnts_col = cnts[:, :N_NODES, 0:1]
    return _combine(sums, cnts_col)


# speedup vs baseline: 3.6475x; 1.1780x over previous
import functools

import jax
import jax.numpy as jnp
from jax import lax
from jax.experimental import pallas as pl
from jax.experimental.pallas import tpu as pltpu, tpu_sc as plsc

N_NODES = 10000
N_EDGES = 320000
D_FEAT = 128

_NC = 2
_NS = 16
_NW = _NC * _NS
_LANES = 16

_CHUNK = N_EDGES // _NW
_K = 80
_NB = 126
_CHUNK_PAD = _NB * _K
_N_PAD = 10112
_DUMP_ROW = _N_PAD - 1
_ROWS_PER_TILE = _N_PAD // _NS

_mesh = plsc.VectorSubcoreMesh(core_axis_name="c", subcore_axis_name="s")


def _fill_2d(ref, nrows, ncols, val):
    v = jnp.full((_LANES,), val, jnp.float32)

    def row(i, _):
        for j in range(ncols // _LANES):
            ref[i, pl.ds(j * _LANES, _LANES)] = v
        return 0

    lax.fori_loop(0, nrows, row, 0)


@functools.partial(
    pl.kernel,
    out_type=(
        jax.ShapeDtypeStruct((_NC, _N_PAD, D_FEAT), jnp.float32),
        jax.ShapeDtypeStruct((_NC, _N_PAD, D_FEAT), jnp.float32),
    ),
    mesh=_mesh,
    scratch_types=(
        pltpu.VMEM((_CHUNK_PAD,), jnp.int32),
        pltpu.VMEM((_NB, _K), jnp.int32),
        pltpu.VMEM((_K, D_FEAT), jnp.float32),
        pltpu.VMEM((_K, D_FEAT), jnp.float32),
        pltpu.VMEM_SHARED((_N_PAD, D_FEAT), jnp.float32),
        pltpu.SemaphoreType.DMA,
        pltpu.SemaphoreType.DMA,
        pltpu.SemaphoreType.DMA,
        pltpu.SemaphoreType.DMA,
    ),
)
def _phase1(feat_hbm, nidx_hbm, seg_hbm, sums_out, cnts_out,
            idx_v, seg_v, rows0, rows1, acc, gs0, gs1, ss0, ss1):
    cid = lax.axis_index("c")
    sid = lax.axis_index("s")
    wid = cid * _NS + sid
    base = wid * _CHUNK_PAD
    r0 = sid * _ROWS_PER_TILE
    nzb = _ROWS_PER_TILE // _K
    nzt = _ROWS_PER_TILE - nzb * _K
    rows = (rows0, rows1)
    gsem = (gs0, gs1)
    ssem = (ss0, ss1)

    def zero_acc():
        for i in range(nzb):
            pltpu.sync_copy(rows0, acc.at[pl.ds(r0 + i * _K, _K)])
        pltpu.sync_copy(rows0.at[pl.ds(0, nzt)],
                        acc.at[pl.ds(r0 + nzb * _K, nzt)])

    def gather_src(g):
        return feat_hbm.at[idx_v.at[pl.ds(g * _K, _K)]]

    def start_gather(g, b):
        pltpu.async_copy(gather_src(g), rows[b], gsem[b])

    def wait_gather(g, b):
        pltpu.make_async_copy(gather_src(g), rows[b], gsem[b]).wait()

    def start_scatter(g, b):
        pltpu.async_copy(rows[b], acc.at[seg_v.at[g]], ssem[b], add=True)

    def wait_scatter(b):
        pltpu.make_async_copy(rows[b], acc.at[seg_v.at[0]], ssem[b]).wait()

    pltpu.sync_copy(nidx_hbm.at[pl.ds(base, _CHUNK_PAD)], idx_v)
    pltpu.sync_copy(seg_hbm.at[wid], seg_v)

    _fill_2d(rows0, _K, D_FEAT, 0.0)
    zero_acc()
    plsc.subcore_barrier()

    start_gather(0, 0)

    def pair_a(i, _):
        for b in range(2):
            g = 2 * i + b
            b1 = (b + 1) % 2
            pred = jnp.logical_and(g >= 1, g + 1 < _NB)

            @pl.when(pred)
            def _():
                wait_scatter(b1)

            @pl.when(g + 1 < _NB)
            def _():
                start_gather(g + 1, b1)

            wait_gather(g, b)
            start_scatter(g, b)
        return 0

    lax.fori_loop(0, _NB // 2, pair_a, 0)
    wait_scatter(0)
    wait_scatter(1)
    plsc.subcore_barrier()
    pltpu.sync_copy(acc.at[pl.ds(r0, _ROWS_PER_TILE)],
                    sums_out.at[cid, pl.ds(r0, _ROWS_PER_TILE)])
    plsc.subcore_barrier()

    _fill_2d(rows0, _K, D_FEAT, 0.0)
    zero_acc()
    _fill_2d(rows0, _K, D_FEAT, 1.0)
    plsc.subcore_barrier()

    def start_scatter_b(g, b):
        pltpu.async_copy(rows0, acc.at[seg_v.at[g]], ssem[b], add=True)

    def wait_scatter_b(b):
        pltpu.make_async_copy(rows0, acc.at[seg_v.at[0]], ssem[b]).wait()

    def pair_b(i, _):
        for b in range(2):
            g = 2 * i + b

            @pl.when(g >= 2)
            def _():
                wait_scatter_b(b)

            start_scatter_b(g, b)
        return 0

    lax.fori_loop(0, _NB // 2, pair_b, 0)
    wait_scatter_b(0)
    wait_scatter_b(1)
    plsc.subcore_barrier()
    pltpu.sync_copy(acc.at[pl.ds(r0, _ROWS_PER_TILE)],
                    cnts_out.at[cid, pl.ds(r0, _ROWS_PER_TILE)])


_BLK = 2000


def _combine_body(sums_ref, cnts_ref, out_ref):
    s = sums_ref[0] + sums_ref[1]
    c = cnts_ref[0] + cnts_ref[1]
    out_ref[...] = jnp.where(c > 0.0, s / jnp.maximum(c, 1.0), 0.0)


_combine = pl.pallas_call(
    _combine_body,
    grid=(N_NODES // _BLK,),
    in_specs=[
        pl.BlockSpec((_NC, _BLK, D_FEAT), lambda i: (0, i, 0)),
        pl.BlockSpec((_NC, _BLK, 1), lambda i: (0, i, 0)),
    ],
    out_specs=pl.BlockSpec((_BLK, D_FEAT), lambda i: (i, 0)),
    out_shape=jax.ShapeDtypeStruct((N_NODES, D_FEAT), jnp.float32),
)


def kernel(features, neighbor_idx, segment_ids, num_samples):
    del num_samples
    pad = _CHUNK_PAD - _CHUNK
    nidx = jnp.pad(neighbor_idx.reshape(_NW, _CHUNK), ((0, 0), (0, pad)),
                   constant_values=0).reshape(_NW * _CHUNK_PAD)
    seg = jnp.pad(segment_ids.reshape(_NW, _CHUNK), ((0, 0), (0, pad)),
                  constant_values=_DUMP_ROW).reshape(_NW, _NB, _K)
    sums, cnts = _phase1(features, nidx, seg)
    cnts_col = cnts[:, :N_NODES, 0:1]
    return _combine(sums, cnts_col)


# speedup vs baseline: 4.6078x; 1.2633x over previous
import functools

import jax
import jax.numpy as jnp
from jax import lax
from jax.experimental import pallas as pl
from jax.experimental.pallas import tpu as pltpu, tpu_sc as plsc

N_NODES = 10000
N_EDGES = 320000
D_FEAT = 128

_NC = 2
_NS = 16
_LANES = 16

_EPT = N_EDGES // _NS
_K = 80
_NBH = 125
_HALF = _NBH * _K
_N_PAD = 10112
_ROWS_PER_TILE = _N_PAD // _NS

_mesh = plsc.VectorSubcoreMesh(core_axis_name="c", subcore_axis_name="s")


def _fill_2d(ref, nrows, ncols, val):
    v = jnp.full((_LANES,), val, jnp.float32)

    def row(i, _):
        for j in range(ncols // _LANES):
            ref[i, pl.ds(j * _LANES, _LANES)] = v
        return 0

    lax.fori_loop(0, nrows, row, 0)


@functools.partial(
    pl.kernel,
    out_type=(
        jax.ShapeDtypeStruct((_N_PAD, D_FEAT), jnp.float32),
        jax.ShapeDtypeStruct((_N_PAD, D_FEAT), jnp.float32),
    ),
    mesh=_mesh,
    scratch_types=(
        pltpu.VMEM((_HALF,), jnp.int32),
        pltpu.VMEM((_NBH, _K), jnp.int32),
        pltpu.VMEM((_K, D_FEAT), jnp.float32),
        pltpu.VMEM((_K, D_FEAT), jnp.float32),
        pltpu.VMEM_SHARED((_N_PAD, D_FEAT), jnp.float32),
        pltpu.SemaphoreType.DMA,
        pltpu.SemaphoreType.DMA,
        pltpu.SemaphoreType.DMA,
        pltpu.SemaphoreType.DMA,
    ),
)
def _phase1(feat_hbm, nidx_hbm, seg_hbm, sums_out, cnts_out,
            idx_v, seg_v, rows0, rows1, acc, gs0, gs1, ss0, ss1):
    cid = lax.axis_index("c")
    sid = lax.axis_index("s")
    r0 = sid * _ROWS_PER_TILE
    nzb = _ROWS_PER_TILE // _K
    nzt = _ROWS_PER_TILE - nzb * _K
    rows = (rows0, rows1)
    gsem = (gs0, gs1)
    ssem = (ss0, ss1)

    def zero_acc():
        for i in range(nzb):
            pltpu.sync_copy(rows0, acc.at[pl.ds(r0 + i * _K, _K)])
        pltpu.sync_copy(rows0.at[pl.ds(0, nzt)],
                        acc.at[pl.ds(r0 + nzb * _K, nzt)])

    def gather_src(g):
        return feat_hbm.at[idx_v.at[pl.ds(g * _K, _K)]]

    def start_gather(g, b):
        pltpu.async_copy(gather_src(g), rows[b], gsem[b])

    def wait_gather(g, b):
        pltpu.make_async_copy(gather_src(g), rows[b], gsem[b]).wait()

    def start_scatter(g, b):
        pltpu.async_copy(rows[b], acc.at[seg_v.at[g]], ssem[b], add=True)

    def wait_scatter(b):
        pltpu.make_async_copy(rows[b], acc.at[seg_v.at[0]], ssem[b]).wait()

    def start_scatter_ones(g, b):
        pltpu.async_copy(rows0, acc.at[seg_v.at[g]], ssem[b], add=True)

    def wait_scatter_ones(b):
        pltpu.make_async_copy(rows0, acc.at[seg_v.at[0]], ssem[b]).wait()

    @pl.when(cid == 0)
    def _():
        _fill_2d(rows0, _K, D_FEAT, 0.0)
        zero_acc()
        plsc.subcore_barrier()

        for half in range(2):
            base = sid * _EPT + half * _HALF
            pltpu.sync_copy(nidx_hbm.at[pl.ds(base, _HALF)], idx_v)
            pltpu.sync_copy(seg_hbm.at[2 * sid + half], seg_v)

            start_gather(0, 0)

            def pair_a(i, _):
                for b in range(2):
                    g = 2 * i + b
                    b1 = (b + 1) % 2
                    pred = jnp.logical_and(g >= 1, g + 1 < _NBH)

                    @pl.when(pred)
                    def _():
                        wait_scatter(b1)

                    @pl.when(g + 1 < _NBH)
                    def _():
                        start_gather(g + 1, b1)

                    wait_gather(g, b)
                    start_scatter(g, b)
                return 0

            lax.fori_loop(0, _NBH // 2, pair_a, 0)
            wait_gather(_NBH - 1, 0)
            start_scatter(_NBH - 1, 0)
            wait_scatter(1)
            wait_scatter(0)

        plsc.subcore_barrier()
        pltpu.sync_copy(acc.at[pl.ds(r0, _ROWS_PER_TILE)],
                        sums_out.at[pl.ds(r0, _ROWS_PER_TILE)])

    @pl.when(cid == 1)
    def _():
        _fill_2d(rows0, _K, D_FEAT, 0.0)
        zero_acc()
        _fill_2d(rows0, _K, D_FEAT, 1.0)
        plsc.subcore_barrier()

        for half in range(2):
            pltpu.sync_copy(seg_hbm.at[2 * sid + half], seg_v)

            def pair_b(i, _):
                for b in range(2):
                    g = 2 * i + b

                    @pl.when(g >= 2)
                    def _():
                        wait_scatter_ones(b)

                    start_scatter_ones(g, b)
                return 0

            lax.fori_loop(0, _NBH // 2, pair_b, 0)
            wait_scatter_ones(0)
            start_scatter_ones(_NBH - 1, 0)
            wait_scatter_ones(1)
            wait_scatter_ones(0)

        plsc.subcore_barrier()
        pltpu.sync_copy(acc.at[pl.ds(r0, _ROWS_PER_TILE)],
                        cnts_out.at[pl.ds(r0, _ROWS_PER_TILE)])


_BLK = 2000


def _combine_body(sums_ref, cnts_ref, out_ref):
    s = sums_ref[...]
    c = cnts_ref[...]
    out_ref[...] = jnp.where(c > 0.0, s / jnp.maximum(c, 1.0), 0.0)


_combine = pl.pallas_call(
    _combine_body,
    grid=(N_NODES // _BLK,),
    in_specs=[
        pl.BlockSpec((_BLK, D_FEAT), lambda i: (i, 0)),
        pl.BlockSpec((_BLK, 1), lambda i: (i, 0)),
    ],
    out_specs=pl.BlockSpec((_BLK, D_FEAT), lambda i: (i, 0)),
    out_shape=jax.ShapeDtypeStruct((N_NODES, D_FEAT), jnp.float32),
)


def kernel(features, neighbor_idx, segment_ids, num_samples):
    del num_samples
    seg = segment_ids.reshape(_NS * 2, _NBH, _K)
    sums, cnts = _phase1(features, neighbor_idx, seg)
    cnts_col = cnts[:N_NODES, 0:1]
    return _combine(sums, cnts_col)
